# baseline (speedup 1.0000x reference)
import functools

import jax
import jax.numpy as jnp
from jax import lax
from jax.experimental import pallas as pl
from jax.experimental.pallas import tpu as pltpu
from jax.experimental.pallas import tpu_sc as plsc

_B = 1024
_LEN = 200
_D = 64
_NC, _NS = 2, 16
_NW = _NC * _NS
_SPW = _B // _NW
_HB = 16
_CH = 24
_NCH = 9
_CHUNKS = tuple((c * _CH, _CH if c < 8 else _LEN - 8 * _CH)
                for c in range(_NCH))
_L16 = 16


def _sc_body(xt, table, pre, out, xblk, idx_v, stg_v, asm_v, cls_v, sep_v,
             gsem, osem):
    c = lax.axis_index("c")
    s = lax.axis_index("s")
    wid = s * _NC + c
    base = wid * _SPW

    pltpu.sync_copy(xt.at[:, pl.ds(base, _SPW)], xblk)
    pltpu.sync_copy(pre.at[0], cls_v)
    pltpu.sync_copy(pre.at[1], sep_v)

    lanes = lax.iota(jnp.int32, _L16)

    def xpose_x(p, carry):
        ch = p // _CH
        pp = p % _CH
        for q0 in (0, 16):
            v = xblk[p, pl.ds(q0, _L16)]
            plsc.store_scatter(
                idx_v,
                [q0 + lanes, jnp.full((_L16,), ch, jnp.int32),
                 jnp.full((_L16,), pp, jnp.int32)],
                v)
        return carry

    lax.fori_loop(0, _LEN, xpose_x, 0)

    def gather_descs(u, sl):
        ch, h = u // 2, u % 2
        _, ln = _CHUNKS[ch]
        return tuple(
            pltpu.make_async_copy(
                table.at[idx_v.at[h * _HB + qg, ch, pl.ds(0, ln)]],
                stg_v.at[sl, qg, pl.ds(0, ln)],
                gsem.at[sl])
            for qg in range(_HB)
        )

    tb = wid // 4
    bc0 = (wid % 4) * _SPW

    def out_desc(u, sl):
        ch, h = u // 2, u % 2
        p0, ln = _CHUNKS[ch]
        return pltpu.make_async_copy(
            asm_v.at[sl, pl.ds(0, ln)],
            out.at[pl.ds(p0, ln), :, tb, :, pl.ds(bc0 + h * _HB, _HB)],
            osem.at[sl])

    hi3 = lax.shift_right_logical(lanes, 3)
    lo3 = lanes & 7

    def xpose_unit(u, sl):
        ch = u // 2
        _p0, ln = _CHUNKS[ch]

        def put(pp_vec, qg, d0, src):
            plsc.store_scatter(
                asm_v.at[sl],
                [pp_vec, 2 * d0 + hi3, lo3,
                 jnp.full((_L16,), qg, jnp.int32)],
                src)

        def body(pp, carry):
            pp_vec = jnp.full((_L16,), pp, jnp.int32)
            for qg in range(_HB):
                for d0 in range(_D // _L16):
                    put(pp_vec, qg, d0,
                        stg_v[sl, qg, pp, pl.ds(d0 * _L16, _L16)])
            return carry

        lo = 1 if ch == 0 else 0
        hi = ln - 1 if ch == _NCH - 1 else ln
        lax.fori_loop(lo, hi, body, 0)
        fix = []
        if ch == 0:
            fix.append((0, cls_v))
        if ch == _NCH - 1:
            fix.append((ln - 1, sep_v))
        for pp, vec in fix:
            pp_vec = jnp.full((_L16,), pp, jnp.int32)
            for qg in range(_HB):
                for d0 in range(_D // _L16):
                    put(pp_vec, qg, d0, vec[pl.ds(d0 * _L16, _L16)])

    _NU = 2 * _NCH
    for u in (0, 1):
        for d in gather_descs(u, u % 2):
            d.start()

    for u in range(_NU):
        sl = u % 2
        if u >= 2:
            out_desc(u - 2, sl).wait()
        for d in gather_descs(u, sl):
            d.wait()
        xpose_unit(u, sl)
        out_desc(u, sl).start()
        if u + 2 < _NU:
            for d in gather_descs(u + 2, sl):
                d.start()

    for u in (_NU - 2, _NU - 1):
        out_desc(u, u % 2).wait()


_gather_call = functools.partial(
    pl.kernel,
    out_type=jax.ShapeDtypeStruct((_LEN, _D // 8, _B // 128, 8, 128),
                                  jnp.float32),
    mesh=plsc.VectorSubcoreMesh(core_axis_name="c", subcore_axis_name="s"),
    compiler_params=pltpu.CompilerParams(use_tc_tiling_on_sc=False,
                                         needs_layout_passes=False),
    scratch_types=[
        pltpu.VMEM((_LEN, _SPW), jnp.int32),
        pltpu.VMEM((_SPW, _NCH, _CH), jnp.int32),
        pltpu.VMEM((2, _HB, _CH, _D), jnp.float32),
        pltpu.VMEM((2, _CH, _D // 8, 8, _HB), jnp.float32),
        pltpu.VMEM((_D,), jnp.float32),
        pltpu.VMEM((_D,), jnp.float32),
        pltpu.SemaphoreType.DMA((2,)),
        pltpu.SemaphoreType.DMA((2,)),
    ],
)(_sc_body)


@jax.jit
def kernel(x, node_table, pre_table):
    out5 = _gather_call(x.T.astype(jnp.int32), node_table, pre_table)
    return out5.transpose(2, 4, 0, 1, 3).reshape(_B, _LEN, _D)
